# baseline (speedup 1.0000x reference)
import functools

import jax
import jax.numpy as jnp
from jax import lax
from jax.experimental import pallas as pl
from jax.experimental.pallas import tpu as pltpu
from jax.experimental.pallas import tpu_sc as plsc

N = 10000
E = 320000
DIM = 128
NLAYER = 6
CUTOFF = 5.0
NRBF = 16

NW = 32
CP = 128
CE = 32
N_PAD = 10240
E_PAD = 327680
EPW = E_PAD // NW
NCHUNK_P = EPW // CP
NCHUNK_E = EPW // CE
STRIPE = N_PAD // 16

_mesh = plsc.VectorSubcoreMesh(core_axis_name="c", subcore_axis_name="s",
                               num_cores=2, num_subcores=16)


@functools.partial(
    pl.kernel,
    out_type=jax.ShapeDtypeStruct((E_PAD,), jnp.float32),
    mesh=_mesh,
    compiler_params=pltpu.CompilerParams(needs_layout_passes=False),
    scratch_types=[
        pltpu.VMEM((EPW,), jnp.int32),
        pltpu.VMEM((EPW,), jnp.int32),
        pltpu.VMEM((EPW,), jnp.float32),
        pltpu.VMEM((N,), jnp.float32),
        pltpu.VMEM((N,), jnp.float32),
        pltpu.VMEM((N,), jnp.float32),
    ],
)
def _sc_prep(px_hbm, py_hbm, pz_hbm, row_hbm, col_hbm, d2_hbm,
             ir, ic, d2b, px, py, pz):
    wid = lax.axis_index("s") * 2 + lax.axis_index("c")
    base = wid * EPW
    pltpu.sync_copy(px_hbm, px)
    pltpu.sync_copy(py_hbm, py)
    pltpu.sync_copy(pz_hbm, pz)
    pltpu.sync_copy(row_hbm.at[pl.ds(base, EPW)], ir)
    pltpu.sync_copy(col_hbm.at[pl.ds(base, EPW)], ic)

    @plsc.parallel_loop(0, EPW // 16, 1, unroll=4)
    def sub(t):
        sl = pl.ds(t * 16, 16)
        idr = ir[sl]
        idc = ic[sl]
        dx = plsc.load_gather(px, [idr]) - plsc.load_gather(px, [idc])
        dy = plsc.load_gather(py, [idr]) - plsc.load_gather(py, [idc])
        dz = plsc.load_gather(pz, [idr]) - plsc.load_gather(pz, [idc])
        d2b[sl] = dx * dx + dy * dy + dz * dz
    pltpu.sync_copy(d2b, d2_hbm.at[pl.ds(base, EPW)])


@functools.partial(
    pl.kernel,
    out_type=jax.ShapeDtypeStruct((2 * N_PAD, DIM), jnp.float32),
    mesh=_mesh,
    compiler_params=pltpu.CompilerParams(needs_layout_passes=False),
    scratch_types=[
        pltpu.VMEM((EPW,), jnp.int32),
        pltpu.VMEM((EPW,), jnp.int32),
        [pltpu.VMEM((CE,), jnp.int32)] * 2,
        [pltpu.VMEM((CE,), jnp.int32)] * 2,
        [pltpu.VMEM((CE, DIM), jnp.float32)] * 2,
        [pltpu.VMEM((CE, DIM), jnp.float32)] * 2,
        [pltpu.VMEM((CE, DIM), jnp.float32)] * 2,
        pltpu.VMEM_SHARED((N_PAD, DIM), jnp.float32),
        [pltpu.SemaphoreType.DMA] * 2,
        [pltpu.SemaphoreType.DMA] * 2,
        [pltpu.SemaphoreType.DMA] * 2,
        [pltpu.SemaphoreType.DMA] * 2,
    ],
)
def _sc_edge(p_hbm, q_hbm, rbf_hbm, row_hbm, col_hbm, out_hbm,
             ir, ic, irc, icc, pg, qg, rb, agg, sp, sq, sr, ss):
    cid = lax.axis_index("c")
    sid = lax.axis_index("s")
    wid = sid * 2 + cid
    ebase = wid * EPW

    pltpu.sync_copy(row_hbm.at[pl.ds(ebase, EPW)], ir)
    pltpu.sync_copy(col_hbm.at[pl.ds(ebase, EPW)], ic)

    def zrow(i, carry):
        for j in range(8):
            pg[0][i, pl.ds(j * 16, 16)] = jnp.zeros((16,), jnp.float32)
        return carry

    lax.fori_loop(0, CE, zrow, 0)
    for c2 in range(STRIPE // CE):
        pltpu.sync_copy(pg[0], agg.at[pl.ds(sid * STRIPE + c2 * CE, CE)])
    plsc.subcore_barrier()

    def issue(k, b):
        for j in range(CE // 16):
            irc[b][pl.ds(j * 16, 16)] = ir[pl.ds(k * CE + j * 16, 16)]
            icc[b][pl.ds(j * 16, 16)] = ic[pl.ds(k * CE + j * 16, 16)]
        pltpu.async_copy(p_hbm.at[irc[b]], pg[b], sp[b])
        pltpu.async_copy(q_hbm.at[icc[b]], qg[b], sq[b])
        pltpu.async_copy(rbf_hbm.at[pl.ds(ebase + k * CE, CE)], rb[b], sr[b])

    def wait_gathers(b):
        pltpu.make_async_copy(p_hbm.at[irc[b]], pg[b], sp[b]).wait()
        pltpu.make_async_copy(q_hbm.at[icc[b]], qg[b], sq[b]).wait()
        pltpu.make_async_copy(rbf_hbm.at[pl.ds(0, CE)], rb[b], sr[b]).wait()

    def wait_scatter(b):
        pltpu.make_async_copy(pg[b], agg.at[icc[b]], ss[b]).wait()

    issue(0, 0)

    def pair(g, carry):
        for b in range(2):
            k = 2 * g + b
            nb = 1 - b
            wait_gathers(b)

            @pl.when(k > 0)
            def _():
                wait_scatter(nb)

            @pl.when(k + 1 < NCHUNK_E)
            def _():
                issue(k + 1, nb)

            @plsc.parallel_loop(0, CE, 1, unroll=4)
            def rowfn(i):
                for j in range(8):
                    sl = pl.ds(j * 16, 16)
                    s = pg[b][i, sl] + qg[b][i, sl]
                    m = s * rb[b][i, sl] / (1.0 + jnp.exp(-s))
                    pg[b][i, sl] = m
            pltpu.async_copy(pg[b], agg.at[icc[b]], ss[b], add=True)
        return carry

    lax.fori_loop(0, NCHUNK_E // 2, pair, 0)
    wait_scatter(1)
    plsc.subcore_barrier()

    for c2 in range(STRIPE // CE):
        r0 = sid * STRIPE + c2 * CE
        pltpu.sync_copy(agg.at[pl.ds(r0, CE)], pg[0])
        pltpu.sync_copy(pg[0], out_hbm.at[pl.ds(cid * N_PAD + r0, CE)])


_NB = N_PAD // 1024


def _t_embed(z2, emb8):
    def body(z_ref, e_ref, o_ref):
        zb = z_ref[...]
        oh = (zb == lax.broadcasted_iota(jnp.int32, (1024, 8), 1)
              ).astype(jnp.float32)
        o_ref[...] = jnp.dot(oh, e_ref[...], preferred_element_type=jnp.float32)

    return pl.pallas_call(
        body,
        grid=(_NB,),
        in_specs=[pl.BlockSpec((1024, 1), lambda i: (i, 0)),
                  pl.BlockSpec((8, DIM), lambda i: (0, 0))],
        out_specs=pl.BlockSpec((1024, DIM), lambda i: (i, 0)),
        out_shape=jax.ShapeDtypeStruct((N_PAD, DIM), jnp.float32),
    )(z2, emb8)


_EB = 4096


def _t_rbf(dist2, w, b):
    def body(d2_ref, w_ref, b_ref, o_ref):
        i = pl.program_id(0)
        dist = jnp.sqrt(d2_ref[...] + 1e-12) + 1e-6
        d = dist / CUTOFF
        d2 = d * d
        d4 = d2 * d2
        d5 = d4 * d
        d6 = d4 * d2
        d7 = d6 * d
        env = 1.0 / d - 28.0 * d5 + 48.0 * d6 - 21.0 * d7
        freq = ((lax.broadcasted_iota(jnp.int32, (1, NRBF), 1) + 1)
                .astype(jnp.float32) * jnp.pi)
        rbf = env * jnp.sin(freq * d)
        x = jnp.dot(rbf, w_ref[...], preferred_element_type=jnp.float32) + b_ref[...]
        y = x / (1.0 + jnp.exp(-x))
        rowid = i * _EB + lax.broadcasted_iota(jnp.int32, (_EB, 1), 0)
        o_ref[...] = jnp.where(rowid < E, y, 0.0)

    return pl.pallas_call(
        body,
        grid=(E_PAD // _EB,),
        in_specs=[pl.BlockSpec((_EB, 1), lambda i: (i, 0)),
                  pl.BlockSpec((NRBF, DIM), lambda i: (0, 0)),
                  pl.BlockSpec((1, DIM), lambda i: (0, 0))],
        out_specs=pl.BlockSpec((_EB, DIM), lambda i: (i, 0)),
        out_shape=jax.ShapeDtypeStruct((E_PAD, DIM), jnp.float32),
    )(dist2, w, b)


def _t_pq(h, wa, wb, b1l):
    def body(h_ref, wa_ref, wb_ref, b_ref, p_ref, q_ref):
        hb = h_ref[...]
        p_ref[...] = jnp.dot(hb, wa_ref[...], preferred_element_type=jnp.float32)
        q_ref[...] = (jnp.dot(hb, wb_ref[...], preferred_element_type=jnp.float32)
                      + b_ref[...])

    return pl.pallas_call(
        body,
        grid=(_NB,),
        in_specs=[pl.BlockSpec((1024, DIM), lambda i: (i, 0)),
                  pl.BlockSpec((DIM, DIM), lambda i: (0, 0)),
                  pl.BlockSpec((DIM, DIM), lambda i: (0, 0)),
                  pl.BlockSpec((1, DIM), lambda i: (0, 0))],
        out_specs=[pl.BlockSpec((1024, DIM), lambda i: (i, 0)),
                   pl.BlockSpec((1024, DIM), lambda i: (i, 0))],
        out_shape=[jax.ShapeDtypeStruct((N_PAD, DIM), jnp.float32),
                   jax.ShapeDtypeStruct((N_PAD, DIM), jnp.float32)],
    )(h, wa, wb, b1l)


def _t_upd(h, a0, a1, w2l, b2l):
    def body(h_ref, a0_ref, a1_ref, w_ref, b_ref, o_ref):
        a = a0_ref[...] + a1_ref[...]
        x = jnp.dot(a, w_ref[...], preferred_element_type=jnp.float32) + b_ref[...]
        o_ref[...] = h_ref[...] + x / (1.0 + jnp.exp(-x))

    return pl.pallas_call(
        body,
        grid=(_NB,),
        in_specs=[pl.BlockSpec((1024, DIM), lambda i: (i, 0)),
                  pl.BlockSpec((1024, DIM), lambda i: (i, 0)),
                  pl.BlockSpec((1024, DIM), lambda i: (i, 0)),
                  pl.BlockSpec((DIM, DIM), lambda i: (0, 0)),
                  pl.BlockSpec((1, DIM), lambda i: (0, 0))],
        out_specs=pl.BlockSpec((1024, DIM), lambda i: (i, 0)),
        out_shape=jax.ShapeDtypeStruct((N_PAD, DIM), jnp.float32),
    )(h, a0, a1, w2l, b2l)


def _t_out(h, wt, b0):
    def body(h_ref, w_ref, b_ref, o_ref, acc):
        i = pl.program_id(0)

        @pl.when(i == 0)
        def _():
            acc[...] = jnp.zeros((1, DIM), jnp.float32)

        acc[...] += jnp.sum(h_ref[...], axis=0, keepdims=True)

        @pl.when(i == pl.num_programs(0) - 1)
        def _():
            o_ref[...] = (jnp.sum(acc[...] * w_ref[...], axis=1, keepdims=True)
                          + N * b_ref[...])

    return pl.pallas_call(
        body,
        grid=(10,),
        in_specs=[pl.BlockSpec((1000, DIM), lambda i: (i, 0)),
                  pl.BlockSpec((1, DIM), lambda i: (0, 0)),
                  pl.BlockSpec((1, 1), lambda i: (0, 0))],
        out_specs=pl.BlockSpec((1, 1), lambda i: (0, 0)),
        out_shape=jax.ShapeDtypeStruct((1, 1), jnp.float32),
        scratch_shapes=[pltpu.VMEM((1, DIM), jnp.float32)],
    )(h, wt, b0)


def kernel(z, pos, edge_index, embeddings, W_rbf, b_rbf, W1, b1, W2, b2,
           W_out, b_out):
    row = edge_index[0].astype(jnp.int32)
    col = edge_index[1].astype(jnp.int32)
    row_p = jnp.concatenate([row, jnp.zeros((E_PAD - E,), jnp.int32)])
    col_p = jnp.concatenate([col, jnp.zeros((E_PAD - E,), jnp.int32)])
    posf = pos.astype(jnp.float32)

    dist2 = _sc_prep(posf[:, 0], posf[:, 1], posf[:, 2], row_p, col_p)
    rbf_h = _t_rbf(dist2.reshape(E_PAD, 1), W_rbf, b_rbf.reshape(1, DIM))

    z2 = jnp.pad(z.astype(jnp.int32), (0, N_PAD - N)).reshape(N_PAD, 1)
    emb8 = jnp.pad(embeddings, ((0, 3), (0, 0)))
    h = _t_embed(z2, emb8)

    for l in range(NLAYER):
        P, Q = _t_pq(h, W1[l, :DIM, :], W1[l, DIM:, :], b1[l].reshape(1, DIM))
        aggf = _sc_edge(P, Q, rbf_h, row_p, col_p)
        h = _t_upd(h, aggf[:N_PAD], aggf[N_PAD:], W2[l], b2[l].reshape(1, DIM))

    out = _t_out(h, W_out.reshape(1, DIM), b_out.reshape(1, 1))
    return out.reshape(1)


# speedup vs baseline: 1.0201x; 1.0201x over previous
import functools

import jax
import jax.numpy as jnp
from jax import lax
from jax.experimental import pallas as pl
from jax.experimental.pallas import tpu as pltpu
from jax.experimental.pallas import tpu_sc as plsc

N = 10000
E = 320000
DIM = 128
NLAYER = 6
CUTOFF = 5.0
NRBF = 16

NW = 32
CP = 128
CE = 64
IB = 16
N_PAD = 10240
E_PAD = 327680
EPW = E_PAD // NW
NCHUNK_P = EPW // CP
NCHUNK_E = EPW // CE
STRIPE = N_PAD // 16

_mesh = plsc.VectorSubcoreMesh(core_axis_name="c", subcore_axis_name="s",
                               num_cores=2, num_subcores=16)


@functools.partial(
    pl.kernel,
    out_type=jax.ShapeDtypeStruct((E_PAD,), jnp.float32),
    mesh=_mesh,
    compiler_params=pltpu.CompilerParams(needs_layout_passes=False),
    scratch_types=[
        pltpu.VMEM((EPW,), jnp.int32),
        pltpu.VMEM((EPW,), jnp.int32),
        pltpu.VMEM((EPW,), jnp.float32),
        pltpu.VMEM((N,), jnp.float32),
        pltpu.VMEM((N,), jnp.float32),
        pltpu.VMEM((N,), jnp.float32),
    ],
)
def _sc_prep(px_hbm, py_hbm, pz_hbm, row_hbm, col_hbm, d2_hbm,
             ir, ic, d2b, px, py, pz):
    wid = lax.axis_index("s") * 2 + lax.axis_index("c")
    base = wid * EPW
    pltpu.sync_copy(px_hbm, px)
    pltpu.sync_copy(py_hbm, py)
    pltpu.sync_copy(pz_hbm, pz)
    pltpu.sync_copy(row_hbm.at[pl.ds(base, EPW)], ir)
    pltpu.sync_copy(col_hbm.at[pl.ds(base, EPW)], ic)

    @plsc.parallel_loop(0, EPW // 16, 1, unroll=4)
    def sub(t):
        sl = pl.ds(t * 16, 16)
        idr = ir[sl]
        idc = ic[sl]
        dx = plsc.load_gather(px, [idr]) - plsc.load_gather(px, [idc])
        dy = plsc.load_gather(py, [idr]) - plsc.load_gather(py, [idc])
        dz = plsc.load_gather(pz, [idr]) - plsc.load_gather(pz, [idc])
        d2b[sl] = dx * dx + dy * dy + dz * dz
    pltpu.sync_copy(d2b, d2_hbm.at[pl.ds(base, EPW)])


@functools.partial(
    pl.kernel,
    out_type=jax.ShapeDtypeStruct((2 * N_PAD, DIM), jnp.float32),
    mesh=_mesh,
    compiler_params=pltpu.CompilerParams(needs_layout_passes=False),
    scratch_types=[
        pltpu.VMEM((IB * CE,), jnp.int32),
        pltpu.VMEM((IB * CE,), jnp.int32),
        [pltpu.VMEM((CE,), jnp.int32)] * 2,
        [pltpu.VMEM((CE,), jnp.int32)] * 2,
        [pltpu.VMEM((CE, DIM), jnp.float32)] * 2,
        [pltpu.VMEM((CE, DIM), jnp.float32)] * 2,
        pltpu.VMEM_SHARED((N_PAD, DIM), jnp.float32),
        [pltpu.SemaphoreType.DMA] * 2,
        [pltpu.SemaphoreType.DMA] * 2,
        [pltpu.SemaphoreType.DMA] * 2,
        [pltpu.SemaphoreType.DMA] * 2,
    ],
)
def _sc_edge(p_hbm, q_hbm, rbf_hbm, row_hbm, col_hbm, out_hbm,
             irb, icb, irc, icc, pg, rb, agg, sp, sq, sr, ss):
    cid = lax.axis_index("c")
    sid = lax.axis_index("s")
    wid = sid * 2 + cid
    ebase = wid * EPW

    def load_iblock(blk):
        pltpu.sync_copy(row_hbm.at[pl.ds(ebase + blk * IB * CE, IB * CE)], irb)
        pltpu.sync_copy(col_hbm.at[pl.ds(ebase + blk * IB * CE, IB * CE)], icb)

    def zero_buf(buf):
        @plsc.parallel_loop(0, CE, 1, unroll=4)
        def _(i):
            for j in range(8):
                buf[i, pl.ds(j * 16, 16)] = jnp.zeros((16,), jnp.float32)

    zero_buf(pg[0])
    for c2 in range(STRIPE // CE):
        pltpu.sync_copy(pg[0], agg.at[pl.ds(sid * STRIPE + c2 * CE, CE)])
    plsc.subcore_barrier()

    def issue(k, b):
        pos = lax.rem(k, IB) * CE
        for j in range(CE // 16):
            irc[b][pl.ds(j * 16, 16)] = irb[pl.ds(pos + j * 16, 16)]
            icc[b][pl.ds(j * 16, 16)] = icb[pl.ds(pos + j * 16, 16)]
        zero_buf(pg[b])
        pltpu.async_copy(p_hbm.at[irc[b]], pg[b], sp[b], add=True)
        pltpu.async_copy(q_hbm.at[icc[b]], pg[b], sq[b], add=True)
        pltpu.async_copy(rbf_hbm.at[pl.ds(ebase + k * CE, CE)], rb[b], sr[b])

    def wait_gathers(b):
        pltpu.make_async_copy(p_hbm.at[irc[b]], pg[b], sp[b]).wait()
        pltpu.make_async_copy(q_hbm.at[icc[b]], pg[b], sq[b]).wait()
        pltpu.make_async_copy(rbf_hbm.at[pl.ds(0, CE)], rb[b], sr[b]).wait()

    def wait_scatter(b):
        pltpu.make_async_copy(pg[b], agg.at[icc[b]], ss[b]).wait()

    load_iblock(0)
    issue(0, 0)

    def pair(g, carry):
        for b in range(2):
            k = 2 * g + b
            nb = 1 - b
            wait_gathers(b)

            @pl.when(k > 0)
            def _():
                wait_scatter(nb)

            @pl.when(jnp.logical_and(lax.rem(k + 1, IB) == 0,
                                     k + 1 < NCHUNK_E))
            def _():
                load_iblock((k + 1) // IB)

            @pl.when(k + 1 < NCHUNK_E)
            def _():
                issue(k + 1, nb)

            @plsc.parallel_loop(0, CE, 1, unroll=4)
            def rowfn(i):
                for j in range(8):
                    sl = pl.ds(j * 16, 16)
                    s = pg[b][i, sl]
                    m = s * rb[b][i, sl] / (1.0 + jnp.exp(-s))
                    pg[b][i, sl] = m

            pltpu.async_copy(pg[b], agg.at[icc[b]], ss[b], add=True)
        return carry

    lax.fori_loop(0, NCHUNK_E // 2, pair, 0)
    wait_scatter(1)
    plsc.subcore_barrier()

    for c2 in range(STRIPE // CE):
        r0 = sid * STRIPE + c2 * CE
        pltpu.sync_copy(agg.at[pl.ds(r0, CE)], pg[0])
        pltpu.sync_copy(pg[0], out_hbm.at[pl.ds(cid * N_PAD + r0, CE)])


_NB = N_PAD // 1024


def _t_embed(z2, emb8):
    def body(z_ref, e_ref, o_ref):
        zb = z_ref[...]
        oh = (zb == lax.broadcasted_iota(jnp.int32, (1024, 8), 1)
              ).astype(jnp.float32)
        o_ref[...] = jnp.dot(oh, e_ref[...], preferred_element_type=jnp.float32)

    return pl.pallas_call(
        body,
        grid=(_NB,),
        in_specs=[pl.BlockSpec((1024, 1), lambda i: (i, 0)),
                  pl.BlockSpec((8, DIM), lambda i: (0, 0))],
        out_specs=pl.BlockSpec((1024, DIM), lambda i: (i, 0)),
        out_shape=jax.ShapeDtypeStruct((N_PAD, DIM), jnp.float32),
    )(z2, emb8)


_EB = 4096


def _t_rbf(dist2, w, b):
    def body(d2_ref, w_ref, b_ref, o_ref):
        i = pl.program_id(0)
        dist = jnp.sqrt(d2_ref[...] + 1e-12) + 1e-6
        d = dist / CUTOFF
        d2 = d * d
        d4 = d2 * d2
        d5 = d4 * d
        d6 = d4 * d2
        d7 = d6 * d
        env = 1.0 / d - 28.0 * d5 + 48.0 * d6 - 21.0 * d7
        freq = ((lax.broadcasted_iota(jnp.int32, (1, NRBF), 1) + 1)
                .astype(jnp.float32) * jnp.pi)
        rbf = env * jnp.sin(freq * d)
        x = jnp.dot(rbf, w_ref[...], preferred_element_type=jnp.float32) + b_ref[...]
        y = x / (1.0 + jnp.exp(-x))
        rowid = i * _EB + lax.broadcasted_iota(jnp.int32, (_EB, 1), 0)
        o_ref[...] = jnp.where(rowid < E, y, 0.0)

    return pl.pallas_call(
        body,
        grid=(E_PAD // _EB,),
        in_specs=[pl.BlockSpec((_EB, 1), lambda i: (i, 0)),
                  pl.BlockSpec((NRBF, DIM), lambda i: (0, 0)),
                  pl.BlockSpec((1, DIM), lambda i: (0, 0))],
        out_specs=pl.BlockSpec((_EB, DIM), lambda i: (i, 0)),
        out_shape=jax.ShapeDtypeStruct((E_PAD, DIM), jnp.float32),
    )(dist2, w, b)


def _t_pq(h, wa, wb, b1l):
    def body(h_ref, wa_ref, wb_ref, b_ref, p_ref, q_ref):
        hb = h_ref[...]
        p_ref[...] = jnp.dot(hb, wa_ref[...], preferred_element_type=jnp.float32)
        q_ref[...] = (jnp.dot(hb, wb_ref[...], preferred_element_type=jnp.float32)
                      + b_ref[...])

    return pl.pallas_call(
        body,
        grid=(_NB,),
        in_specs=[pl.BlockSpec((1024, DIM), lambda i: (i, 0)),
                  pl.BlockSpec((DIM, DIM), lambda i: (0, 0)),
                  pl.BlockSpec((DIM, DIM), lambda i: (0, 0)),
                  pl.BlockSpec((1, DIM), lambda i: (0, 0))],
        out_specs=[pl.BlockSpec((1024, DIM), lambda i: (i, 0)),
                   pl.BlockSpec((1024, DIM), lambda i: (i, 0))],
        out_shape=[jax.ShapeDtypeStruct((N_PAD, DIM), jnp.float32),
                   jax.ShapeDtypeStruct((N_PAD, DIM), jnp.float32)],
    )(h, wa, wb, b1l)


def _t_upd(h, a0, a1, w2l, b2l):
    def body(h_ref, a0_ref, a1_ref, w_ref, b_ref, o_ref):
        a = a0_ref[...] + a1_ref[...]
        x = jnp.dot(a, w_ref[...], preferred_element_type=jnp.float32) + b_ref[...]
        o_ref[...] = h_ref[...] + x / (1.0 + jnp.exp(-x))

    return pl.pallas_call(
        body,
        grid=(_NB,),
        in_specs=[pl.BlockSpec((1024, DIM), lambda i: (i, 0)),
                  pl.BlockSpec((1024, DIM), lambda i: (i, 0)),
                  pl.BlockSpec((1024, DIM), lambda i: (i, 0)),
                  pl.BlockSpec((DIM, DIM), lambda i: (0, 0)),
                  pl.BlockSpec((1, DIM), lambda i: (0, 0))],
        out_specs=pl.BlockSpec((1024, DIM), lambda i: (i, 0)),
        out_shape=jax.ShapeDtypeStruct((N_PAD, DIM), jnp.float32),
    )(h, a0, a1, w2l, b2l)


def _t_out(h, wt, b0):
    def body(h_ref, w_ref, b_ref, o_ref, acc):
        i = pl.program_id(0)

        @pl.when(i == 0)
        def _():
            acc[...] = jnp.zeros((1, DIM), jnp.float32)

        acc[...] += jnp.sum(h_ref[...], axis=0, keepdims=True)

        @pl.when(i == pl.num_programs(0) - 1)
        def _():
            o_ref[...] = (jnp.sum(acc[...] * w_ref[...], axis=1, keepdims=True)
                          + N * b_ref[...])

    return pl.pallas_call(
        body,
        grid=(10,),
        in_specs=[pl.BlockSpec((1000, DIM), lambda i: (i, 0)),
                  pl.BlockSpec((1, DIM), lambda i: (0, 0)),
                  pl.BlockSpec((1, 1), lambda i: (0, 0))],
        out_specs=pl.BlockSpec((1, 1), lambda i: (0, 0)),
        out_shape=jax.ShapeDtypeStruct((1, 1), jnp.float32),
        scratch_shapes=[pltpu.VMEM((1, DIM), jnp.float32)],
    )(h, wt, b0)


def kernel(z, pos, edge_index, embeddings, W_rbf, b_rbf, W1, b1, W2, b2,
           W_out, b_out):
    row = edge_index[0].astype(jnp.int32)
    col = edge_index[1].astype(jnp.int32)
    row_p = jnp.concatenate([row, jnp.zeros((E_PAD - E,), jnp.int32)])
    col_p = jnp.concatenate([col, jnp.zeros((E_PAD - E,), jnp.int32)])
    posf = pos.astype(jnp.float32)

    dist2 = _sc_prep(posf[:, 0], posf[:, 1], posf[:, 2], row_p, col_p)
    rbf_h = _t_rbf(dist2.reshape(E_PAD, 1), W_rbf, b_rbf.reshape(1, DIM))

    z2 = jnp.pad(z.astype(jnp.int32), (0, N_PAD - N)).reshape(N_PAD, 1)
    emb8 = jnp.pad(embeddings, ((0, 3), (0, 0)))
    h = _t_embed(z2, emb8)

    for l in range(NLAYER):
        P, Q = _t_pq(h, W1[l, :DIM, :], W1[l, DIM:, :], b1[l].reshape(1, DIM))
        aggf = _sc_edge(P, Q, rbf_h, row_p, col_p)
        h = _t_upd(h, aggf[:N_PAD], aggf[N_PAD:], W2[l], b2[l].reshape(1, DIM))

    out = _t_out(h, W_out.reshape(1, DIM), b_out.reshape(1, 1))
    return out.reshape(1)


# speedup vs baseline: 1.0342x; 1.0138x over previous
import functools

import jax
import jax.numpy as jnp
from jax import lax
from jax.experimental import pallas as pl
from jax.experimental.pallas import tpu as pltpu
from jax.experimental.pallas import tpu_sc as plsc

N = 10000
E = 320000
DIM = 128
NLAYER = 6
CUTOFF = 5.0
NRBF = 16

NW = 32
CP = 128
CE = 64
IB = 16
N_PAD = 10240
E_PAD = 327680
EPW = E_PAD // NW
NCHUNK_P = EPW // CP
NCHUNK_E = EPW // CE
STRIPE = N_PAD // 16

_mesh = plsc.VectorSubcoreMesh(core_axis_name="c", subcore_axis_name="s",
                               num_cores=2, num_subcores=16)


@functools.partial(
    pl.kernel,
    out_type=jax.ShapeDtypeStruct((E_PAD,), jnp.float32),
    mesh=_mesh,
    compiler_params=pltpu.CompilerParams(needs_layout_passes=False),
    scratch_types=[
        pltpu.VMEM((EPW,), jnp.int32),
        pltpu.VMEM((EPW,), jnp.int32),
        pltpu.VMEM((EPW,), jnp.float32),
        pltpu.VMEM((N,), jnp.float32),
        pltpu.VMEM((N,), jnp.float32),
        pltpu.VMEM((N,), jnp.float32),
    ],
)
def _sc_prep(px_hbm, py_hbm, pz_hbm, row_hbm, col_hbm, d2_hbm,
             ir, ic, d2b, px, py, pz):
    wid = lax.axis_index("s") * 2 + lax.axis_index("c")
    base = wid * EPW
    pltpu.sync_copy(px_hbm, px)
    pltpu.sync_copy(py_hbm, py)
    pltpu.sync_copy(pz_hbm, pz)
    pltpu.sync_copy(row_hbm.at[pl.ds(base, EPW)], ir)
    pltpu.sync_copy(col_hbm.at[pl.ds(base, EPW)], ic)

    @plsc.parallel_loop(0, EPW // 16, 1, unroll=4)
    def sub(t):
        sl = pl.ds(t * 16, 16)
        idr = ir[sl]
        idc = ic[sl]
        dx = plsc.load_gather(px, [idr]) - plsc.load_gather(px, [idc])
        dy = plsc.load_gather(py, [idr]) - plsc.load_gather(py, [idc])
        dz = plsc.load_gather(pz, [idr]) - plsc.load_gather(pz, [idc])
        d2b[sl] = dx * dx + dy * dy + dz * dz
    pltpu.sync_copy(d2b, d2_hbm.at[pl.ds(base, EPW)])


@functools.partial(
    pl.kernel,
    out_type=jax.ShapeDtypeStruct((2 * N_PAD, DIM), jnp.float32),
    mesh=_mesh,
    compiler_params=pltpu.CompilerParams(needs_layout_passes=False),
    scratch_types=[
        pltpu.VMEM((IB * CE,), jnp.int32),
        pltpu.VMEM((IB * CE,), jnp.int32),
        [pltpu.VMEM((CE,), jnp.int32)] * 2,
        [pltpu.VMEM((CE,), jnp.int32)] * 2,
        [pltpu.VMEM((CE, DIM), jnp.float32)] * 2,
        [pltpu.VMEM((CE, DIM), jnp.float32)] * 2,
        pltpu.VMEM_SHARED((N_PAD, DIM), jnp.float32),
        [pltpu.SemaphoreType.DMA] * 2,
        [pltpu.SemaphoreType.DMA] * 2,
        [pltpu.SemaphoreType.DMA] * 2,
        [pltpu.SemaphoreType.DMA] * 2,
    ],
)
def _sc_edge(p_hbm, q_hbm, rbf_hbm, row_hbm, col_hbm, out_hbm,
             irb, icb, irc, icc, pg, rb, agg, sp, sq, sr, ss):
    cid = lax.axis_index("c")
    sid = lax.axis_index("s")
    wid = sid * 2 + cid
    ebase = wid * EPW

    def load_iblock(blk):
        pltpu.sync_copy(row_hbm.at[pl.ds(ebase + blk * IB * CE, IB * CE)], irb)
        pltpu.sync_copy(col_hbm.at[pl.ds(ebase + blk * IB * CE, IB * CE)], icb)

    def zero_buf(buf):
        @plsc.parallel_loop(0, CE, 1, unroll=4)
        def _(i):
            for j in range(8):
                buf[i, pl.ds(j * 16, 16)] = jnp.zeros((16,), jnp.float32)

    @plsc.parallel_loop(0, CE, 1, unroll=4)
    def _zr(i):
        for j in range(8):
            rb[0][i, pl.ds(j * 16, 16)] = jnp.zeros((16,), jnp.float32)
    for c2 in range(STRIPE // CE):
        pltpu.sync_copy(rb[0], agg.at[pl.ds(sid * STRIPE + c2 * CE, CE)])
    plsc.subcore_barrier()

    def issue(k, b):
        pos = lax.rem(k, IB) * CE
        for j in range(CE // 16):
            irc[b][pl.ds(j * 16, 16)] = irb[pl.ds(pos + j * 16, 16)]
            icc[b][pl.ds(j * 16, 16)] = icb[pl.ds(pos + j * 16, 16)]
        zero_buf(pg[b])
        pltpu.async_copy(p_hbm.at[irc[b]], pg[b], sp[b], add=True)
        pltpu.async_copy(q_hbm.at[icc[b]], pg[b], sq[b], add=True)
        pltpu.async_copy(rbf_hbm.at[pl.ds(ebase + k * CE, CE)], rb[b], sr[b])

    def wait_gathers(b):
        pltpu.make_async_copy(p_hbm.at[irc[b]], pg[b], sp[b]).wait()
        pltpu.make_async_copy(q_hbm.at[icc[b]], pg[b], sq[b]).wait()
        pltpu.make_async_copy(rbf_hbm.at[pl.ds(0, CE)], rb[b], sr[b]).wait()

    def wait_scatter(b):
        pltpu.make_async_copy(pg[b], agg.at[icc[b]], ss[b]).wait()

    load_iblock(0)
    issue(0, 0)

    def pair(g, carry):
        for b in range(2):
            k = 2 * g + b
            nb = 1 - b
            wait_gathers(b)

            @pl.when(k > 0)
            def _():
                wait_scatter(nb)

            @pl.when(jnp.logical_and(lax.rem(k + 1, IB) == 0,
                                     k + 1 < NCHUNK_E))
            def _():
                load_iblock((k + 1) // IB)

            @pl.when(k + 1 < NCHUNK_E)
            def _():
                issue(k + 1, nb)

            @plsc.parallel_loop(0, CE, 1, unroll=4)
            def rowfn(i):
                for j in range(8):
                    sl = pl.ds(j * 16, 16)
                    s = pg[b][i, sl]
                    m = s * rb[b][i, sl] / (1.0 + jnp.exp(-s))
                    pg[b][i, sl] = m

            pltpu.async_copy(pg[b], agg.at[icc[b]], ss[b], add=True)
        return carry

    lax.fori_loop(0, NCHUNK_E // 2, pair, 0)
    wait_scatter(1)
    plsc.subcore_barrier()

    for c2 in range(STRIPE // CE):
        r0 = sid * STRIPE + c2 * CE
        pltpu.sync_copy(agg.at[pl.ds(r0, CE)], rb[0])
        pltpu.sync_copy(rb[0], out_hbm.at[pl.ds(cid * N_PAD + r0, CE)])


_NB = N_PAD // 1024


def _t_embed(z2, emb8):
    def body(z_ref, e_ref, o_ref):
        zb = z_ref[...]
        oh = (zb == lax.broadcasted_iota(jnp.int32, (1024, 8), 1)
              ).astype(jnp.float32)
        o_ref[...] = jnp.dot(oh, e_ref[...], preferred_element_type=jnp.float32)

    return pl.pallas_call(
        body,
        grid=(_NB,),
        in_specs=[pl.BlockSpec((1024, 1), lambda i: (i, 0)),
                  pl.BlockSpec((8, DIM), lambda i: (0, 0))],
        out_specs=pl.BlockSpec((1024, DIM), lambda i: (i, 0)),
        out_shape=jax.ShapeDtypeStruct((N_PAD, DIM), jnp.float32),
    )(z2, emb8)


_EB = 4096


def _t_rbf(dist2, w, b):
    def body(d2_ref, w_ref, b_ref, o_ref):
        i = pl.program_id(0)
        dist = jnp.sqrt(d2_ref[...] + 1e-12) + 1e-6
        d = dist / CUTOFF
        d2 = d * d
        d4 = d2 * d2
        d5 = d4 * d
        d6 = d4 * d2
        d7 = d6 * d
        env = 1.0 / d - 28.0 * d5 + 48.0 * d6 - 21.0 * d7
        freq = ((lax.broadcasted_iota(jnp.int32, (1, NRBF), 1) + 1)
                .astype(jnp.float32) * jnp.pi)
        rbf = env * jnp.sin(freq * d)
        x = jnp.dot(rbf, w_ref[...], preferred_element_type=jnp.float32) + b_ref[...]
        y = x / (1.0 + jnp.exp(-x))
        rowid = i * _EB + lax.broadcasted_iota(jnp.int32, (_EB, 1), 0)
        o_ref[...] = jnp.where(rowid < E, y, 0.0)

    return pl.pallas_call(
        body,
        grid=(E_PAD // _EB,),
        in_specs=[pl.BlockSpec((_EB, 1), lambda i: (i, 0)),
                  pl.BlockSpec((NRBF, DIM), lambda i: (0, 0)),
                  pl.BlockSpec((1, DIM), lambda i: (0, 0))],
        out_specs=pl.BlockSpec((_EB, DIM), lambda i: (i, 0)),
        out_shape=jax.ShapeDtypeStruct((E_PAD, DIM), jnp.float32),
    )(dist2, w, b)


def _t_pq(h, wa, wb, b1l):
    def body(h_ref, wa_ref, wb_ref, b_ref, p_ref, q_ref):
        hb = h_ref[...]
        p_ref[...] = jnp.dot(hb, wa_ref[...], preferred_element_type=jnp.float32)
        q_ref[...] = (jnp.dot(hb, wb_ref[...], preferred_element_type=jnp.float32)
                      + b_ref[...])

    return pl.pallas_call(
        body,
        grid=(_NB,),
        in_specs=[pl.BlockSpec((1024, DIM), lambda i: (i, 0)),
                  pl.BlockSpec((DIM, DIM), lambda i: (0, 0)),
                  pl.BlockSpec((DIM, DIM), lambda i: (0, 0)),
                  pl.BlockSpec((1, DIM), lambda i: (0, 0))],
        out_specs=[pl.BlockSpec((1024, DIM), lambda i: (i, 0)),
                   pl.BlockSpec((1024, DIM), lambda i: (i, 0))],
        out_shape=[jax.ShapeDtypeStruct((N_PAD, DIM), jnp.float32),
                   jax.ShapeDtypeStruct((N_PAD, DIM), jnp.float32)],
    )(h, wa, wb, b1l)


def _t_upd(h, a0, a1, w2l, b2l):
    def body(h_ref, a0_ref, a1_ref, w_ref, b_ref, o_ref):
        a = a0_ref[...] + a1_ref[...]
        x = jnp.dot(a, w_ref[...], preferred_element_type=jnp.float32) + b_ref[...]
        o_ref[...] = h_ref[...] + x / (1.0 + jnp.exp(-x))

    return pl.pallas_call(
        body,
        grid=(_NB,),
        in_specs=[pl.BlockSpec((1024, DIM), lambda i: (i, 0)),
                  pl.BlockSpec((1024, DIM), lambda i: (i, 0)),
                  pl.BlockSpec((1024, DIM), lambda i: (i, 0)),
                  pl.BlockSpec((DIM, DIM), lambda i: (0, 0)),
                  pl.BlockSpec((1, DIM), lambda i: (0, 0))],
        out_specs=pl.BlockSpec((1024, DIM), lambda i: (i, 0)),
        out_shape=jax.ShapeDtypeStruct((N_PAD, DIM), jnp.float32),
    )(h, a0, a1, w2l, b2l)


def _t_upq(h, a0, a1, w2l, b2l, wa, wb, b1n):
    def body(h_ref, a0_ref, a1_ref, w_ref, b_ref, wa_ref, wb_ref, bn_ref,
             o_ref, p_ref, q_ref):
        a = a0_ref[...] + a1_ref[...]
        x = jnp.dot(a, w_ref[...], preferred_element_type=jnp.float32) + b_ref[...]
        hn = h_ref[...] + x / (1.0 + jnp.exp(-x))
        o_ref[...] = hn
        p_ref[...] = jnp.dot(hn, wa_ref[...], preferred_element_type=jnp.float32)
        q_ref[...] = (jnp.dot(hn, wb_ref[...], preferred_element_type=jnp.float32)
                      + bn_ref[...])

    return pl.pallas_call(
        body,
        grid=(_NB,),
        in_specs=[pl.BlockSpec((1024, DIM), lambda i: (i, 0)),
                  pl.BlockSpec((1024, DIM), lambda i: (i, 0)),
                  pl.BlockSpec((1024, DIM), lambda i: (i, 0)),
                  pl.BlockSpec((DIM, DIM), lambda i: (0, 0)),
                  pl.BlockSpec((1, DIM), lambda i: (0, 0)),
                  pl.BlockSpec((DIM, DIM), lambda i: (0, 0)),
                  pl.BlockSpec((DIM, DIM), lambda i: (0, 0)),
                  pl.BlockSpec((1, DIM), lambda i: (0, 0))],
        out_specs=[pl.BlockSpec((1024, DIM), lambda i: (i, 0)),
                   pl.BlockSpec((1024, DIM), lambda i: (i, 0)),
                   pl.BlockSpec((1024, DIM), lambda i: (i, 0))],
        out_shape=[jax.ShapeDtypeStruct((N_PAD, DIM), jnp.float32),
                   jax.ShapeDtypeStruct((N_PAD, DIM), jnp.float32),
                   jax.ShapeDtypeStruct((N_PAD, DIM), jnp.float32)],
    )(h, a0, a1, w2l, b2l, wa, wb, b1n)


def _t_out(h, wt, b0):
    def body(h_ref, w_ref, b_ref, o_ref, acc):
        i = pl.program_id(0)

        @pl.when(i == 0)
        def _():
            acc[...] = jnp.zeros((1, DIM), jnp.float32)

        acc[...] += jnp.sum(h_ref[...], axis=0, keepdims=True)

        @pl.when(i == pl.num_programs(0) - 1)
        def _():
            o_ref[...] = (jnp.sum(acc[...] * w_ref[...], axis=1, keepdims=True)
                          + N * b_ref[...])

    return pl.pallas_call(
        body,
        grid=(10,),
        in_specs=[pl.BlockSpec((1000, DIM), lambda i: (i, 0)),
                  pl.BlockSpec((1, DIM), lambda i: (0, 0)),
                  pl.BlockSpec((1, 1), lambda i: (0, 0))],
        out_specs=pl.BlockSpec((1, 1), lambda i: (0, 0)),
        out_shape=jax.ShapeDtypeStruct((1, 1), jnp.float32),
        scratch_shapes=[pltpu.VMEM((1, DIM), jnp.float32)],
    )(h, wt, b0)


def kernel(z, pos, edge_index, embeddings, W_rbf, b_rbf, W1, b1, W2, b2,
           W_out, b_out):
    row = edge_index[0].astype(jnp.int32)
    col = edge_index[1].astype(jnp.int32)
    row_p = jnp.concatenate([row, jnp.zeros((E_PAD - E,), jnp.int32)])
    col_p = jnp.concatenate([col, jnp.zeros((E_PAD - E,), jnp.int32)])
    posf = pos.astype(jnp.float32)

    dist2 = _sc_prep(posf[:, 0], posf[:, 1], posf[:, 2], row_p, col_p)
    rbf_h = _t_rbf(dist2.reshape(E_PAD, 1), W_rbf, b_rbf.reshape(1, DIM))

    z2 = jnp.pad(z.astype(jnp.int32), (0, N_PAD - N)).reshape(N_PAD, 1)
    emb8 = jnp.pad(embeddings, ((0, 3), (0, 0)))
    h = _t_embed(z2, emb8)

    P, Q = _t_pq(h, W1[0, :DIM, :], W1[0, DIM:, :], b1[0].reshape(1, DIM))
    for l in range(NLAYER):
        aggf = _sc_edge(P, Q, rbf_h, row_p, col_p)
        if l + 1 < NLAYER:
            h, P, Q = _t_upq(h, aggf[:N_PAD], aggf[N_PAD:], W2[l],
                             b2[l].reshape(1, DIM), W1[l + 1, :DIM, :],
                             W1[l + 1, DIM:, :], b1[l + 1].reshape(1, DIM))
        else:
            h = _t_upd(h, aggf[:N_PAD], aggf[N_PAD:], W2[l],
                       b2[l].reshape(1, DIM))

    out = _t_out(h, W_out.reshape(1, DIM), b_out.reshape(1, 1))
    return out.reshape(1)
